# baseline (device time: 299796 ns/iter reference)
import jax
import jax.numpy as jnp
from jax import lax
from jax.experimental import pallas as pl
from jax.experimental.pallas import tpu as pltpu

BLK = 512


def kernel(x, W):
    t, d = x.shape
    _, v = W.shape
    nblk = v // BLK
    half = nblk // 2

    x_bf = x.astype(jnp.bfloat16)

    def body(x_ref, w_ref, out_ref, ry_ref, rz_ref,
             e_loc, w_buf, stage, eld, s_ref, s_rem_ref,
             ys_sems, yr_sems, zs_sems, zr_sems,
             s_send_sem, s_recv_sem, w_sems, stage_sems, eld_sems):
        my_x = lax.axis_index("x")
        my_y = lax.axis_index("y")
        my_z = lax.axis_index("z")
        p_y = (my_x, 1 - my_y, my_z)
        p_z = (my_x, my_y, 1 - my_z)

        barrier = pltpu.get_barrier_semaphore()
        for nbr in (p_y, p_z):
            pl.semaphore_signal(barrier, inc=1, device_id=nbr,
                                device_id_type=pl.DeviceIdType.MESH)
        pl.semaphore_wait(barrier, 2)

        base_send = my_z * half
        base_keep = (1 - my_z) * half

        def blk_idx(pos):
            if pos < half:
                return base_send + pos
            return base_keep + (pos - half)

        def w_load(pos):
            j = blk_idx(pos)
            cp = pltpu.make_async_copy(
                w_ref.at[:, pl.ds(j * BLK, BLK)], w_buf.at[pos % 2],
                w_sems.at[pos % 2])
            cp.start()
            return cp

        def y_rdma(k):
            return pltpu.make_async_remote_copy(
                src_ref=e_loc.at[k], dst_ref=ry_ref.at[k],
                send_sem=ys_sems.at[k], recv_sem=yr_sems.at[k],
                device_id=p_y, device_id_type=pl.DeviceIdType.MESH)

        def z_rdma(k):
            return pltpu.make_async_remote_copy(
                src_ref=ry_ref.at[k], dst_ref=rz_ref.at[k],
                send_sem=zs_sems.at[k], recv_sem=zr_sems.at[k],
                device_id=p_z, device_id_type=pl.DeviceIdType.MESH)

        w_cp = [w_load(0)]

        def compute_block(pos):
            w_cp[0].wait()
            if pos + 1 < nblk:
                w_cp[0] = w_load(pos + 1)
            wb = w_buf[pos % 2].astype(jnp.bfloat16)
            logits = jnp.dot(x_ref[...], wb,
                             preferred_element_type=jnp.float32)
            return jnp.exp(logits)

        acc = jnp.zeros((t, BLK), jnp.float32)
        y_rdmas = []
        z_rdmas = []

        for pos in range(half):
            e = compute_block(pos)
            acc = acc + e
            e_loc[pos] = e.astype(jnp.bfloat16)
            r = y_rdma(pos)
            r.start()
            y_rdmas.append(r)

        for k in range(half):
            y_rdmas[k].wait_recv()
            rz = z_rdma(k)
            rz.start()
            z_rdmas.append(rz)
            e = compute_block(half + k)
            acc = acc + e
            e_loc[half + k] = e.astype(jnp.bfloat16)

        s_val = jnp.sum(acc, axis=1, keepdims=True)
        s_ref[...] = jnp.broadcast_to(s_val, s_ref.shape)
        s_rdma = pltpu.make_async_remote_copy(
            src_ref=s_ref, dst_ref=s_rem_ref,
            send_sem=s_send_sem, recv_sem=s_recv_sem,
            device_id=p_y, device_id_type=pl.DeviceIdType.MESH)
        s_rdma.start()
        s_rdma.wait_recv()
        inv = (1.0 / (s_val + s_rem_ref[:, 0:1])).astype(jnp.bfloat16)

        out_cps = [None, None]
        nctr = [0]

        def stage_out2(a, b, col0):
            slot = nctr[0] % 2
            nctr[0] += 1
            if out_cps[slot] is not None:
                out_cps[slot].wait()
            stage[slot, :, 0:BLK] = a
            stage[slot, :, BLK:2 * BLK] = b
            cp = pltpu.make_async_copy(
                stage.at[slot], out_ref.at[:, pl.ds(col0, 2 * BLK)],
                stage_sems.at[slot])
            cp.start()
            out_cps[slot] = cp

        loc0 = my_y * v
        rem0 = (1 - my_y) * v

        for m in range(half):
            i0 = 2 * m
            col = loc0 + blk_idx(i0) * BLK
            stage_out2(e_loc[i0] * inv, e_loc[i0 + 1] * inv, col)

        def rem_col(m):
            if m < 8:
                return rem0 + (base_send + 2 * m) * BLK
            return rem0 + (base_keep + 2 * (m - 8)) * BLK

        def eld_load(m):
            slot = m % 2
            if m < 8:
                src = ry_ref.at[pl.ds(2 * m, 2)]
            else:
                for k in (2 * (m - 8), 2 * (m - 8) + 1):
                    z_rdmas[k].wait_recv()
                src = rz_ref.at[pl.ds(2 * (m - 8), 2)]
            cp = pltpu.make_async_copy(src, eld.at[slot], eld_sems.at[slot])
            cp.start()
            return cp

        ld = eld_load(0)
        for m in range(16):
            ld.wait()
            nld = eld_load(m + 1) if m + 1 < 16 else None
            stage_out2(eld[m % 2, 0] * inv, eld[m % 2, 1] * inv,
                       rem_col(m))
            ld = nld

        for r in y_rdmas:
            r.wait_send()
        for r in z_rdmas:
            r.wait_send()
        s_rdma.wait_send()
        for cp in out_cps:
            if cp is not None:
                cp.wait()

    out, _ry, _rz = pl.pallas_call(
        body,
        out_shape=[
            jax.ShapeDtypeStruct((t, 2 * v), jnp.bfloat16),
            jax.ShapeDtypeStruct((half, t, BLK), jnp.bfloat16),
            jax.ShapeDtypeStruct((half, t, BLK), jnp.bfloat16),
        ],
        in_specs=[
            pl.BlockSpec(memory_space=pltpu.MemorySpace.VMEM),
            pl.BlockSpec(memory_space=pl.ANY),
        ],
        out_specs=[
            pl.BlockSpec(memory_space=pl.ANY),
            pl.BlockSpec(memory_space=pl.ANY),
            pl.BlockSpec(memory_space=pl.ANY),
        ],
        scratch_shapes=[
            pltpu.MemorySpace.VMEM((nblk, t, BLK), jnp.bfloat16),
            pltpu.MemorySpace.VMEM((2, d, BLK), jnp.float32),
            pltpu.MemorySpace.VMEM((2, t, 2 * BLK), jnp.bfloat16),
            pltpu.MemorySpace.VMEM((2, 2, t, BLK), jnp.bfloat16),
            pltpu.MemorySpace.VMEM((t, 128), jnp.float32),
            pltpu.MemorySpace.VMEM((t, 128), jnp.float32),
            pltpu.SemaphoreType.DMA((half,)),
            pltpu.SemaphoreType.DMA((half,)),
            pltpu.SemaphoreType.DMA((half,)),
            pltpu.SemaphoreType.DMA((half,)),
            pltpu.SemaphoreType.DMA,
            pltpu.SemaphoreType.DMA,
            pltpu.SemaphoreType.DMA((2,)),
            pltpu.SemaphoreType.DMA((2,)),
            pltpu.SemaphoreType.DMA((2,)),
        ],
        compiler_params=pltpu.CompilerParams(
            collective_id=0,
            vmem_limit_bytes=63 * 1024 * 1024,
        ),
    )(x_bf, W)
    return out
